# baseline (device time: 31177 ns/iter reference)
import jax
import jax.numpy as jnp
from jax import lax
from jax.experimental import pallas as pl
from jax.experimental.pallas import tpu as pltpu

N_CHUNKS = 8


def kernel(x, pi):
    shard_shape = x.shape
    rows = shard_shape[1] // N_CHUNKS

    def body(x_ref, pi_ref, out_ref, stage_ref, comm_ref,
             copy_sem, send_sem, recv_sem):
        my_x = lax.axis_index("x")
        my_y = lax.axis_index("y")
        my_z = lax.axis_index("z")
        partner = 1 - my_x
        tgt_x = pi_ref[my_x]

        barrier = pltpu.get_barrier_semaphore()
        pl.semaphore_signal(
            barrier, inc=1,
            device_id=(partner, my_y, my_z),
            device_id_type=pl.DeviceIdType.MESH,
        )
        pl.semaphore_wait(barrier, 1)

        def start_fetch(h):
            cp = pltpu.make_async_copy(
                x_ref.at[0, pl.ds(h * rows, rows), :],
                stage_ref.at[h % 2],
                copy_sem.at[h % 2],
            )
            cp.start()
            return cp

        fetches = {0: start_fetch(0)}
        rdmas = []
        for h in range(N_CHUNKS):
            if h + 1 < N_CHUNKS:
                fetches[h + 1] = start_fetch(h + 1)
            fetches[h].wait()
            sl = pl.ds(h * rows, rows)
            comm_ref[0, sl, :] = stage_ref[h % 2].astype(jnp.bfloat16)
            rdma = pltpu.make_async_remote_copy(
                src_ref=comm_ref.at[0, sl, :],
                dst_ref=out_ref.at[0, sl, :],
                send_sem=send_sem.at[h],
                recv_sem=recv_sem.at[h],
                device_id=(tgt_x, my_y, my_z),
                device_id_type=pl.DeviceIdType.MESH,
            )
            rdma.start()
            rdmas.append(rdma)
        for rdma in rdmas:
            rdma.wait()

    return pl.pallas_call(
        body,
        out_shape=jax.ShapeDtypeStruct(shard_shape, jnp.bfloat16),
        in_specs=[
            pl.BlockSpec(memory_space=pl.ANY),
            pl.BlockSpec(memory_space=pltpu.SMEM),
        ],
        out_specs=pl.BlockSpec(memory_space=pl.ANY),
        scratch_shapes=[
            pltpu.VMEM((2, rows, shard_shape[2]), x.dtype),
            pltpu.VMEM(shard_shape, jnp.bfloat16),
            pltpu.SemaphoreType.DMA((2,)),
            pltpu.SemaphoreType.DMA((N_CHUNKS,)),
            pltpu.SemaphoreType.DMA((N_CHUNKS,)),
        ],
        compiler_params=pltpu.CompilerParams(collective_id=0),
    )(x, pi)
